# Initial kernel scaffold; baseline (speedup 1.0000x reference)
import jax
import jax.numpy as jnp
from jax.experimental import pallas as pl


def kernel(user_idx, item_idx, user_emb, item_emb, w1, b1, w2, b2, w3, b3, wf, bf):
    raise NotImplementedError("write your pallas kernel here")


# speedup vs baseline: 1.0268x; 1.0268x over previous
import functools

import jax
import jax.numpy as jnp
from jax.experimental import pallas as pl
from jax.experimental.pallas import tpu as pltpu


def _round_up(x: int, m: int) -> int:
    return ((x + m - 1) // m) * m


def _neumf_kernel(
    u_ref, i_ref,
    w1u_ref, w1i_ref, b1_ref,
    w2_ref, b2_ref, w3_ref, b3_ref,
    wfg_ref, wfm_ref, bf_ref,
    out_ref,
):
    f32 = jnp.float32
    u = u_ref[...].astype(f32)
    it = i_ref[...].astype(f32)
    h = (jnp.dot(u, w1u_ref[...], preferred_element_type=f32)
         + jnp.dot(it, w1i_ref[...], preferred_element_type=f32)
         + b1_ref[...])
    h = jnp.maximum(h, 0.0)
    h = jnp.maximum(jnp.dot(h, w2_ref[...], preferred_element_type=f32) + b2_ref[...], 0.0)
    h = jnp.maximum(jnp.dot(h, w3_ref[...], preferred_element_type=f32) + b3_ref[...], 0.0)
    score = (jnp.sum(u * it * wfg_ref[...], axis=-1, keepdims=True)
             + jnp.sum(h * wfm_ref[...], axis=-1, keepdims=True)
             + bf_ref[...])
    out_ref[...] = jax.nn.sigmoid(score)


def _whole(a):
    return pl.BlockSpec(a.shape, lambda i: (0,) * a.ndim)


@functools.partial(jax.jit, static_argnames=("tile_b",))
def _forward(user_idx, item_idx, user_emb, item_emb,
             w1, b1, w2, b2, w3, b3, wf, bf, *, tile_b: int = 4096):
    B = int(user_idx.shape[0])
    W = user_emb.shape[1]
    half = w1.shape[0] // 2
    mf_dim = W - half
    l1, l3 = w1.shape[1], w3.shape[1]

    zeros_mf = jnp.zeros((mf_dim, l1), jnp.float32)
    w1u_pad = jnp.concatenate([zeros_mf, w1[:half, :]], axis=0)
    w1i_pad = jnp.concatenate([zeros_mf, w1[half:, :]], axis=0)
    wfg_row = jnp.concatenate(
        [wf[:mf_dim, :], jnp.zeros((half, 1), jnp.float32)], axis=0).T
    wfm_row = wf[mf_dim:, :].T

    b_pad = _round_up(B, tile_b)
    pad = b_pad - B
    uidx = jnp.pad(user_idx.astype(jnp.int32), (0, pad))
    iidx = jnp.pad(item_idx.astype(jnp.int32), (0, pad))

    u_rows = jnp.take(user_emb.astype(jnp.bfloat16), uidx, axis=0)
    i_rows = jnp.take(item_emb.astype(jnp.bfloat16), iidx, axis=0)

    num_tiles = b_pad // tile_b
    act_spec = pl.BlockSpec((tile_b, W), lambda i: (i, 0))
    inputs = (u_rows, i_rows, w1u_pad, w1i_pad, b1, w2, b2, w3, b3,
              wfg_row, wfm_row, bf)
    in_specs = [act_spec, act_spec] + [_whole(a) for a in inputs[2:]]
    out = pl.pallas_call(
        _neumf_kernel,
        out_shape=jax.ShapeDtypeStruct((b_pad, 1), jnp.float32),
        grid=(num_tiles,),
        in_specs=in_specs,
        out_specs=pl.BlockSpec((tile_b, 1), lambda i: (i, 0)),
        compiler_params=pltpu.CompilerParams(
            dimension_semantics=("parallel",),
            vmem_limit_bytes=64 * 1024 * 1024,
        ),
    )(*inputs)
    return out[:B]


def kernel(user_idx, item_idx, user_emb, item_emb, w1, b1, w2, b2, w3, b3, wf, bf):
    return _forward(user_idx, item_idx, user_emb, item_emb,
                    w1, b1, w2, b2, w3, b3, wf, bf)


# speedup vs baseline: 2.4726x; 2.4081x over previous
import functools

import jax
import jax.numpy as jnp
from jax.experimental import pallas as pl
from jax.experimental.pallas import tpu as pltpu


def _round_up(x: int, m: int) -> int:
    return ((x + m - 1) // m) * m


def _make_transform_kernel(mf_dim: int):
    def _transform(tab_ref, w_ref, b_ref, s_ref, out_ref):
        f32 = jnp.float32
        g = tab_ref[:, :mf_dim] * s_ref[...]
        m = jnp.dot(tab_ref[:, mf_dim:], w_ref[...],
                    preferred_element_type=f32) + b_ref[...]
        out_ref[...] = jnp.concatenate([g, m], axis=1)
    return _transform


def _make_main_kernel(tile_b: int, mf_dim: int):
    def _main(uidx_ref, iidx_ref,
              tu_ref, ti_ref,
              w2_ref, b2_ref, w3_ref, b3_ref,
              wfm_ref, bf_ref,
              out_ref,
              au_ref, ai_ref):
        f32 = jnp.float32
        for r in range(tile_b):
            au_ref[r, :] = tu_ref[uidx_ref[0, 0, r], 0]
            ai_ref[r, :] = ti_ref[iidx_ref[0, 0, r], 0]
        a = au_ref[...]
        b = ai_ref[...]
        h = jnp.maximum(a[:, mf_dim:] + b[:, mf_dim:], 0.0)
        h = jnp.maximum(
            jnp.dot(h, w2_ref[...], preferred_element_type=f32) + b2_ref[...], 0.0)
        h = jnp.maximum(
            jnp.dot(h, w3_ref[...], preferred_element_type=f32) + b3_ref[...], 0.0)
        s8 = a[:, :mf_dim] * b[:, :mf_dim] + h * wfm_ref[...]
        score = jnp.sum(s8, axis=1, keepdims=True) + bf_ref[...]
        out_ref[...] = jax.nn.sigmoid(score)
    return _main


def _transform_table(tab, w, b_row, s_row, *, row_tile: int):
    n, width = tab.shape
    mf_dim = width - w.shape[0]
    n_pad = _round_up(n, row_tile)
    tab_p = jnp.pad(tab, ((0, n_pad - n), (0, 0)))
    grid = n_pad // row_tile
    out = pl.pallas_call(
        _make_transform_kernel(mf_dim),
        out_shape=jax.ShapeDtypeStruct((n_pad, width), jnp.float32),
        grid=(grid,),
        in_specs=[
            pl.BlockSpec((row_tile, width), lambda i: (i, 0)),
            pl.BlockSpec(w.shape, lambda i: (0, 0)),
            pl.BlockSpec(b_row.shape, lambda i: (0, 0)),
            pl.BlockSpec(s_row.shape, lambda i: (0, 0)),
        ],
        out_specs=pl.BlockSpec((row_tile, width), lambda i: (i, 0)),
        compiler_params=pltpu.CompilerParams(
            dimension_semantics=("parallel",)),
    )(tab_p, w, b_row, s_row)
    return out


@functools.partial(jax.jit, static_argnames=("tile_b",))
def _forward(user_idx, item_idx, user_emb, item_emb,
             w1, b1, w2, b2, w3, b3, wf, bf, *, tile_b: int = 256):
    B = int(user_idx.shape[0])
    U, W = user_emb.shape
    half = w1.shape[0] // 2
    mf_dim = W - half
    l3 = w3.shape[1]

    wf_g = wf[:mf_dim, :].T
    ones_g = jnp.ones((1, mf_dim), jnp.float32)
    zero_b = jnp.zeros_like(b1)
    tu = _transform_table(user_emb, w1[:half, :], zero_b, wf_g, row_tile=1024)
    ti = _transform_table(item_emb, w1[half:, :], b1, ones_g, row_tile=1024)
    tu3 = tu.reshape(tu.shape[0], 1, W)
    ti3 = ti.reshape(ti.shape[0], 1, W)

    wfm_row = wf[mf_dim:, :].T

    b_pad = _round_up(B, tile_b)
    pad = b_pad - B
    uidx = jnp.pad(user_idx.astype(jnp.int32), (0, pad)).reshape(-1, 1, tile_b)
    iidx = jnp.pad(item_idx.astype(jnp.int32), (0, pad)).reshape(-1, 1, tile_b)
    num_tiles = b_pad // tile_b

    idx_spec = pl.BlockSpec((1, 1, tile_b), lambda i: (i, 0, 0),
                            memory_space=pltpu.SMEM)

    def _whole(a):
        return pl.BlockSpec(a.shape, lambda i: (0,) * a.ndim)

    out = pl.pallas_call(
        _make_main_kernel(tile_b, mf_dim),
        out_shape=jax.ShapeDtypeStruct((b_pad, 1), jnp.float32),
        grid=(num_tiles,),
        in_specs=[idx_spec, idx_spec,
                  _whole(tu3), _whole(ti3),
                  _whole(w2), _whole(b2), _whole(w3), _whole(b3),
                  _whole(wfm_row), _whole(bf)],
        out_specs=pl.BlockSpec((tile_b, 1), lambda i: (i, 0)),
        scratch_shapes=[pltpu.VMEM((tile_b, W), jnp.float32),
                        pltpu.VMEM((tile_b, W), jnp.float32)],
        compiler_params=pltpu.CompilerParams(
            dimension_semantics=("parallel",),
            vmem_limit_bytes=64 * 1024 * 1024,
        ),
    )(uidx, iidx, tu3, ti3, w2, b2, w3, b3, wfm_row, bf)
    return out[:B]


def kernel(user_idx, item_idx, user_emb, item_emb, w1, b1, w2, b2, w3, b3, wf, bf):
    return _forward(user_idx, item_idx, user_emb, item_emb,
                    w1, b1, w2, b2, w3, b3, wf, bf)
